# baseline (speedup 1.0000x reference)
import functools

import jax
import jax.numpy as jnp
from jax.experimental import pallas as pl

_N = 126
_S = 8
_W = 128
_L = _S * _W
_EPS = 1e-5
_CHS = [(32, 8), (8, 64), (64, 64), (64, 128), (128, 128), (128, 256),
        (256, 1)]
_PK_LANES = 128


def _pack_layout():
    lay = {}
    r = 0
    lay["xT"] = r; r += 8
    lay["WfcT"] = r; r += 32
    lay["bfc"] = r; r += 32
    for b, (cin, cout) in enumerate(_CHS):
        o8 = max(8, cout)
        for t in range(3):
            lay["W%d_%d" % (b, t)] = r
            r += o8 if cin <= _W else 2 * o8
    lay["rows"] = r
    return lay


_LAY = _pack_layout()


def _fused_kernel(x_ref, pk_ref, out_ref):
    x = x_ref[...]
    pk = pk_ref[...]
    xT = pk[_LAY["xT"]:_LAY["xT"] + 5, :_N]
    wfcT = pk[_LAY["WfcT"]:_LAY["WfcT"] + 32, :5]
    bfc = pk[_LAY["bfc"]:_LAY["bfc"] + 32, :1]
    featT = jnp.dot(wfcT, xT,
                    preferred_element_type=jnp.float32) + bfc

    g = jnp.dot(x, xT, preferred_element_type=jnp.float32)
    aa_col = jnp.sum(x * x, axis=1, keepdims=True)
    aa_row = jnp.sum(xT * xT, axis=0, keepdims=True)
    d2 = (aa_row - 2.0 * g) + aa_col
    dis = jnp.sqrt(jnp.maximum(d2, 0.0))

    row_iota = jax.lax.broadcasted_iota(jnp.int32, (_N, _N), 0)
    col_iota = jax.lax.broadcasted_iota(jnp.int32, (_N, _N), 1)
    zpad = jnp.zeros((_N, _W - _N), dtype=jnp.float32)

    parts = [jnp.where(row_iota == col_iota, 1.0, 0.0), zpad]
    for _ in range(7):
        idx = jnp.argmin(dis, axis=0).reshape(1, _N)
        onehot = row_iota == idx
        parts.append(jnp.where(onehot, 1.0, 0.0))
        parts.append(zpad)
        dis = jnp.where(onehot, jnp.inf, dis)
    M = jnp.concatenate(parts, axis=1)

    conv_in = jnp.dot(featT, M, preferred_element_type=jnp.float32,
                      precision=jax.lax.Precision.HIGHEST)
    act = jnp.concatenate(
        [conv_in, jnp.zeros((_W - 32, _L), jnp.float32)], axis=0)

    lane = jax.lax.broadcasted_iota(jnp.int32, (1, _L), 1)
    vmask = jnp.where(lane % _W < _N, 1.0, 0.0)

    for b, (cin, cout) in enumerate(_CHS):
        o8 = max(8, cout)
        kin = act.shape[0]
        sl = jnp.concatenate([act[:, 1:], jnp.zeros((kin, 1), jnp.float32)],
                             axis=1)
        sr = jnp.concatenate([jnp.zeros((kin, 1), jnp.float32), act[:, :-1]],
                             axis=1)
        c = None
        for t, a in ((0, sr), (1, act), (2, sl)):
            r0 = _LAY["W%d_%d" % (b, t)]
            if kin <= _W:
                wt = pk[r0:r0 + o8, :kin]
                p = jnp.dot(wt, a, preferred_element_type=jnp.float32)
            else:
                p = (jnp.dot(pk[r0:r0 + o8, :], a[:_W, :],
                             preferred_element_type=jnp.float32)
                     + jnp.dot(pk[r0 + o8:r0 + 2 * o8, :], a[_W:, :],
                               preferred_element_type=jnp.float32))
            c = p if c is None else c + p
        tm = c * vmask
        t2 = tm * c
        s0s = [jnp.sum(tm[:, s * _W:(s + 1) * _W], axis=1, keepdims=True)
               for s in range(_S)]
        qs = [jnp.sum(t2[:, s * _W:(s + 1) * _W], axis=1, keepdims=True)
              for s in range(_S)]
        ms = [s0 * (1.0 / _N) for s0 in s0s]
        vs = [q * (1.0 / _N) - m * m for q, m in zip(qs, ms)]
        s1s = [jax.lax.rsqrt(v + _EPS) for v in vs]
        sy = sum(((s0 - _N * m) * s1 for s0, m, s1 in zip(s0s, ms, s1s)),
                 jnp.zeros((o8, 1), jnp.float32))
        sy2 = sum(((_N * v) * (s1 * s1) for v, s1 in zip(vs, s1s)),
                  jnp.zeros((o8, 1), jnp.float32))
        m2 = sy * (1.0 / (_S * _N))
        v2 = sy2 * (1.0 / (_S * _N)) - m2 * m2
        inv2 = jax.lax.rsqrt(v2 + _EPS)
        As = [s1 * inv2 for s1 in s1s]
        Bs = [-(m * s1 + m2) * inv2 for m, s1 in zip(ms, s1s)]
        out = jnp.concatenate(
            [c[:, s * _W:(s + 1) * _W] * As[s] + Bs[s] for s in range(_S)],
            axis=1) * vmask
        if cin == cout:
            out = out + act[:o8, :]
        if o8 < _W:
            out = jnp.concatenate(
                [out, jnp.zeros((_W - o8, _L), jnp.float32)], axis=0)
        act = out

    for s in range(_S):
        out_ref[pl.ds(s, 1), :] = act[0:1, s * _W:(s + 1) * _W]


@functools.partial(jax.jit, static_argnums=())
def kernel(x, Wfc, bfc, W1, b1, W2, b2, W3, b3, W4, b4, W5, b5, W6, b6, W7, b7):
    del b1, b2, b3, b4, b5, b6, b7
    pieces = [
        jnp.pad(x.T, ((0, 3), (0, _PK_LANES - _N))),
        jnp.pad(Wfc.T, ((0, 0), (0, _PK_LANES - 5))),
        jnp.pad(bfc.reshape(-1, 1), ((0, 0), (0, _PK_LANES - 1))),
    ]
    for W in (W1, W2, W3, W4, W5, W6, W7):
        o, i, _ = W.shape
        o8 = max(8, o)
        for t in range(3):
            if i <= _PK_LANES:
                pieces.append(jnp.pad(W[:, :, t],
                                      ((0, o8 - o), (0, _PK_LANES - i))))
            else:
                pieces.append(jnp.pad(W[:, :_PK_LANES, t], ((0, o8 - o), (0, 0))))
                pieces.append(jnp.pad(W[:, _PK_LANES:, t], ((0, o8 - o), (0, 0))))
    pack = jnp.concatenate(pieces, axis=0)

    y = pl.pallas_call(
        _fused_kernel,
        out_shape=jax.ShapeDtypeStruct((_S, _W), jnp.float32),
    )(x, pack)
    return y[:, None, :_N]


# speedup vs baseline: 1.0181x; 1.0181x over previous
import functools

import jax
import jax.numpy as jnp
from jax.experimental import pallas as pl

_N = 126
_S = 8
_W = 128
_L = _S * _W
_EPS = 1e-5
_CHS = [(32, 8), (8, 64), (64, 64), (64, 128), (128, 128), (128, 256),
        (256, 1)]
_PK_LANES = 128


def _pack_layout():
    lay = {}
    r = 0
    lay["xT"] = r; r += 8
    lay["WfcT"] = r; r += 32
    lay["bfc"] = r; r += 32
    for b, (cin, cout) in enumerate(_CHS):
        o8 = max(8, cout)
        for t in range(3):
            lay["W%d_%d" % (b, t)] = r
            r += o8 if cin <= _W else 2 * o8
    lay["rows"] = r
    return lay


_LAY = _pack_layout()


def _fused_kernel(x_ref, pk_ref, out_ref):
    x = x_ref[...]
    pk = pk_ref[...]
    xT = pk[_LAY["xT"]:_LAY["xT"] + 5, :_N]
    wfcT = pk[_LAY["WfcT"]:_LAY["WfcT"] + 32, :5]
    bfc = pk[_LAY["bfc"]:_LAY["bfc"] + 32, :1]
    featT = jnp.dot(wfcT, xT,
                    preferred_element_type=jnp.float32) + bfc

    g = jnp.dot(x, xT, preferred_element_type=jnp.float32)
    aa_col = jnp.sum(x * x, axis=1, keepdims=True)
    aa_row = jnp.sum(xT * xT, axis=0, keepdims=True)
    d2 = (aa_row - 2.0 * g) + aa_col
    dis = jnp.sqrt(jnp.maximum(d2, 0.0))

    row_iota = jax.lax.broadcasted_iota(jnp.int32, (_N, _N), 0)
    col_iota = jax.lax.broadcasted_iota(jnp.int32, (_N, _N), 1)
    zpad = jnp.zeros((_N, _W - _N), dtype=jnp.float32)

    parts = [jnp.where(row_iota == col_iota, 1.0, 0.0), zpad]
    for _ in range(7):
        idx = jnp.argmin(dis, axis=0).reshape(1, _N)
        onehot = row_iota == idx
        parts.append(jnp.where(onehot, 1.0, 0.0))
        parts.append(zpad)
        dis = jnp.where(onehot, jnp.inf, dis)
    M = jnp.concatenate(parts, axis=1)

    conv_in = jnp.dot(featT, M, preferred_element_type=jnp.float32,
                      precision=jax.lax.Precision.HIGHEST)
    act = jnp.concatenate(
        [conv_in, jnp.zeros((_W - 32, _L), jnp.float32)], axis=0)

    lane = jax.lax.broadcasted_iota(jnp.int32, (1, _L), 1)
    vmask = jnp.where(lane % _W < _N, 1.0, 0.0)

    for b, (cin, cout) in enumerate(_CHS):
        o8 = max(8, cout)
        kin = act.shape[0]
        sl = jnp.concatenate([act[:, 1:], jnp.zeros((kin, 1), jnp.float32)],
                             axis=1)
        sr = jnp.concatenate([jnp.zeros((kin, 1), jnp.float32), act[:, :-1]],
                             axis=1)
        c = None
        for t, a in ((0, sr), (1, act), (2, sl)):
            r0 = _LAY["W%d_%d" % (b, t)]
            if kin <= _W:
                wt = pk[r0:r0 + o8, :kin]
                p = jnp.dot(wt, a, preferred_element_type=jnp.float32)
            else:
                p = (jnp.dot(pk[r0:r0 + o8, :], a[:_W, :],
                             preferred_element_type=jnp.float32)
                     + jnp.dot(pk[r0 + o8:r0 + 2 * o8, :], a[_W:, :],
                               preferred_element_type=jnp.float32))
            c = p if c is None else c + p
        tm = c * vmask
        t2 = tm * c
        s0s = [jnp.sum(tm[:, s * _W:(s + 1) * _W], axis=1, keepdims=True)
               for s in range(_S)]
        qs = [jnp.sum(t2[:, s * _W:(s + 1) * _W], axis=1, keepdims=True)
              for s in range(_S)]
        ms = [s0 * (1.0 / _N) for s0 in s0s]
        vs = [q * (1.0 / _N) - m * m for q, m in zip(qs, ms)]
        s1s = [jax.lax.rsqrt(v + _EPS) for v in vs]
        ys = [(c[:, s * _W:(s + 1) * _W] - ms[s]) * s1s[s] for s in range(_S)]
        sy = sum(((s0 - _N * m) * s1 for s0, m, s1 in zip(s0s, ms, s1s)),
                 jnp.zeros((o8, 1), jnp.float32))
        sy2 = sum(((_N * v) * (s1 * s1) for v, s1 in zip(vs, s1s)),
                  jnp.zeros((o8, 1), jnp.float32))
        m2 = sy * (1.0 / (_S * _N))
        v2 = sy2 * (1.0 / (_S * _N)) - m2 * m2
        inv2 = jax.lax.rsqrt(v2 + _EPS)
        out = jnp.concatenate([(y - m2) * inv2 for y in ys], axis=1) * vmask
        if cin == cout:
            out = out + act[:o8, :]
        if o8 < _W:
            out = jnp.concatenate(
                [out, jnp.zeros((_W - o8, _L), jnp.float32)], axis=0)
        act = out

    for s in range(_S):
        out_ref[pl.ds(s, 1), :] = act[0:1, s * _W:(s + 1) * _W]


@functools.partial(jax.jit, static_argnums=())
def kernel(x, Wfc, bfc, W1, b1, W2, b2, W3, b3, W4, b4, W5, b5, W6, b6, W7, b7):
    del b1, b2, b3, b4, b5, b6, b7
    pieces = [
        jnp.pad(x.T, ((0, 3), (0, _PK_LANES - _N))),
        jnp.pad(Wfc.T, ((0, 0), (0, _PK_LANES - 5))),
        jnp.pad(bfc.reshape(-1, 1), ((0, 0), (0, _PK_LANES - 1))),
    ]
    for W in (W1, W2, W3, W4, W5, W6, W7):
        o, i, _ = W.shape
        o8 = max(8, o)
        for t in range(3):
            if i <= _PK_LANES:
                pieces.append(jnp.pad(W[:, :, t],
                                      ((0, o8 - o), (0, _PK_LANES - i))))
            else:
                pieces.append(jnp.pad(W[:, :_PK_LANES, t], ((0, o8 - o), (0, 0))))
                pieces.append(jnp.pad(W[:, _PK_LANES:, t], ((0, o8 - o), (0, 0))))
    pack = jnp.concatenate(pieces, axis=0)

    y = pl.pallas_call(
        _fused_kernel,
        out_shape=jax.ShapeDtypeStruct((_S, _W), jnp.float32),
    )(x, pack)
    return y[:, None, :_N]
